# baseline (speedup 1.0000x reference)
import functools

import jax
import jax.numpy as jnp
from jax import lax
from jax.experimental import pallas as pl
from jax.experimental.pallas import tpu as pltpu
from jax.experimental.pallas import tpu_sc as plsc

N = 10000
E = 320000
D_IN = 128
D_OUT = 4

NC = 2
DC = 2
NS = 16
UNROLL = 5
EPT = E // NS
NBLK = EPT // (16 * UNROLL)
N_PAD = 10240
RPB = N_PAD // NS
FPB = RPB * DC
VE = FPB // 16
FW = N_PAD * DC

_mesh = plsc.VectorSubcoreMesh(
    core_axis_name="c", subcore_axis_name="s", num_cores=NC, num_subcores=NS
)


@functools.partial(
    pl.kernel,
    out_type=[jax.ShapeDtypeStruct((NC, FW), jnp.float32),
              jax.ShapeDtypeStruct((NC, FW), jnp.float32)],
    mesh=_mesh,
    scratch_types=[
        pltpu.VMEM((EPT,), jnp.int32),
        pltpu.VMEM((EPT,), jnp.int32),
        pltpu.VMEM((FW,), jnp.float32),
        pltpu.VMEM((FW,), jnp.float32),
        pltpu.VMEM((FPB,), jnp.float32),
        pltpu.VMEM((FPB,), jnp.float32),
        pltpu.VMEM((FPB,), jnp.float32),
        pltpu.VMEM((FPB,), jnp.float32),
        pltpu.VMEM((NS, FPB), jnp.float32),
        pltpu.VMEM_SHARED((NS, FW), jnp.float32),
        pltpu.VMEM_SHARED((FW,), jnp.float32),
        pltpu.SemaphoreType.DMA,
        pltpu.SemaphoreType.DMA,
    ],
    compiler_params=pltpu.CompilerParams(
        use_tc_tiling_on_sc=False, needs_layout_passes=False),
)
def _sgconv_sc(z_hbm, rows_hbm, cols_hbm, out_hbm, tscr_hbm,
               rowv, colv, tbl, acct, zb, disb, dinvb, tmpb, tmp16,
               psh, tsh, zsem, rsem):
    c = lax.axis_index("c")
    s = lax.axis_index("s")
    fsl = pl.ds(s * FPB, FPB)

    zcp = pltpu.async_copy(z_hbm.at[c, fsl], zb, zsem)
    pltpu.sync_copy(rows_hbm.at[s], rowv)
    pltpu.sync_copy(cols_hbm.at[s], colv)

    half = jnp.full((16,), 0.5, jnp.float32)
    three_half = jnp.full((16,), 1.5, jnp.float32)
    magic = jnp.full((16,), 0x5F3759DF, jnp.int32)
    ones16 = jnp.full((16,), 1.0, jnp.float32)

    def zero_acct():
        def z4(j, carry):
            for u in range(UNROLL):
                acct[pl.ds((j * UNROLL + u) * 16, 16)] = jnp.zeros(
                    (16,), jnp.float32)
            return carry
        lax.fori_loop(0, FW // (16 * UNROLL), z4, 0)

    def exchange_and_gather_partials():
        pltpu.sync_copy(acct, psh.at[s])
        plsc.subcore_barrier()
        cps = [pltpu.async_copy(psh.at[t, fsl], tmp16.at[t], rsem)
               for t in range(NS)]
        for cp in cps:
            cp.wait()
        plsc.subcore_barrier()

    def reduce_into_tmpb(base_buf):
        def red(i, carry):
            ix = pl.ds(i * 16, 16)
            acc = base_buf[ix]
            for t in range(NS):
                acc = acc + tmp16[t, ix]
            tmpb[ix] = acc
            return carry
        lax.fori_loop(0, VE, red, 0)

    zero_acct()

    def deg_blk(j, carry):
        for u in range(UNROLL):
            ix = pl.ds((j * UNROLL + u) * 16, 16)
            cx = colv[ix]
            plsc.addupdate_scatter(acct, [cx], ones16)
            plsc.addupdate_scatter(acct, [cx + 1], ones16)
        return carry

    lax.fori_loop(0, NBLK, deg_blk, 0)
    exchange_and_gather_partials()
    zcp.wait()

    def newton(i, carry):
        ix = pl.ds(i * 16, 16)
        d16 = tmp16[0, ix] + 1.0
        for t in range(1, NS):
            d16 = d16 + tmp16[t, ix]
        h = d16 * half
        yi = magic - lax.shift_right_logical(plsc.bitcast(d16, jnp.int32), 1)
        y = plsc.bitcast(yi, jnp.float32)
        y = y * (three_half - h * y * y)
        y = y * (three_half - h * y * y)
        y = y * (three_half - h * y * y)
        disb[ix] = y
        dinvb[ix] = y * y
        zb[ix] = y * zb[ix]
        return carry

    lax.fori_loop(0, VE, newton, 0)
    pltpu.sync_copy(zb, tscr_hbm.at[c, fsl])
    plsc.subcore_barrier()
    pltpu.sync_copy(tscr_hbm.at[c], tbl)

    def hop():
        zero_acct()

        def blk(j, carry):
            for u in range(UNROLL):
                ix = pl.ds((j * UNROLL + u) * 16, 16)
                rx = rowv[ix]
                cx = colv[ix]
                v0 = plsc.load_gather(tbl, [rx])
                plsc.addupdate_scatter(acct, [cx], v0)
                v1 = plsc.load_gather(tbl, [rx + 1])
                plsc.addupdate_scatter(acct, [cx + 1], v1)
            return carry

        lax.fori_loop(0, NBLK, blk, 0)
        exchange_and_gather_partials()

    hop()
    reduce_into_tmpb(zb)

    def scale_w(i, carry):
        ix = pl.ds(i * 16, 16)
        w16 = tmpb[ix] * dinvb[ix]
        tmpb[ix] = w16
        zb[ix] = w16
        return carry

    lax.fori_loop(0, VE, scale_w, 0)
    pltpu.sync_copy(tmpb, tscr_hbm.at[c, fsl])
    plsc.subcore_barrier()
    pltpu.sync_copy(tscr_hbm.at[c], tbl)

    hop()
    reduce_into_tmpb(zb)

    def scale_out(i, carry):
        ix = pl.ds(i * 16, 16)
        tmpb[ix] = tmpb[ix] * disb[ix]
        return carry

    lax.fori_loop(0, VE, scale_out, 0)
    pltpu.sync_copy(tmpb, out_hbm.at[c, fsl])


def _tc_z(x_ref, w_ref, z_ref):
    z_ref[:N, :] = jnp.dot(x_ref[...], w_ref[...],
                           preferred_element_type=jnp.float32)
    z_ref[N:, :] = jnp.zeros((N_PAD - N, D_OUT), jnp.float32)


def _tc_final(h_ref, b_ref, out_ref):
    o = h_ref[...] + b_ref[...]
    m = jnp.max(o, axis=1, keepdims=True)
    e = jnp.exp(o - m)
    lse = jnp.log(jnp.sum(e, axis=1, keepdims=True))
    out_ref[...] = o - m - lse


def kernel(x, edge_index, W, b):
    f32 = jnp.float32
    rows2 = (edge_index[0] * 2).reshape(NS, EPT)
    cols2 = (edge_index[1] * 2).reshape(NS, EPT)

    z4 = pl.pallas_call(
        _tc_z,
        out_shape=jax.ShapeDtypeStruct((N_PAD, D_OUT), f32),
    )(x, W)
    zs = jnp.transpose(z4.reshape(N_PAD, NC, DC), (1, 0, 2)).reshape(NC, FW)

    h, _ = _sgconv_sc(zs, rows2, cols2)
    h4 = jnp.transpose(h.reshape(NC, N_PAD, DC), (1, 0, 2)).reshape(
        N_PAD, D_OUT)

    out = pl.pallas_call(
        _tc_final,
        out_shape=jax.ShapeDtypeStruct((N_PAD, D_OUT), f32),
    )(h4, b)
    return out[:N]
